# Initial kernel scaffold; baseline (speedup 1.0000x reference)
import jax
import jax.numpy as jnp
from jax.experimental import pallas as pl


def kernel(op_idx, stats, pred_flags, col_ids, col_stats, op_table, col_table, W1, b1, W2, b2, Wc, bc, Wo, bo):
    raise NotImplementedError("write your pallas kernel here")


# speedup vs baseline: 1.9577x; 1.9577x over previous
import functools

import jax
import jax.numpy as jnp
from jax import lax
from jax.experimental import pallas as pl
from jax.experimental.pallas import tpu as pltpu
from jax.experimental.pallas import tpu_sc as plsc

_B, _C = 16384, 8
_OP_VOCAB, _OP_DIM = 64, 32
_COL_DIM = 16
_STATS_H, _PRED_DIM, _CSTAT_DIM, _OUT_DIM = 16, 8, 8, 64
_BLK = 1024
_GRID = _B // _BLK


@functools.lru_cache(maxsize=None)
def _build_colmean():
    info = plsc.get_sparse_core_info()
    nc, ns = info.num_cores, info.num_subcores
    nw = nc * ns
    idx_w = _B * _C // nw
    row_w = _B // nw

    mesh = plsc.VectorSubcoreMesh(core_axis_name="c", subcore_axis_name="s")

    @functools.partial(
        pl.kernel,
        mesh=mesh,
        out_type=jax.ShapeDtypeStruct((_B, _COL_DIM), jnp.float32),
        compiler_params=pltpu.CompilerParams(use_tc_tiling_on_sc=False),
        scratch_types=[
            pltpu.VMEM((idx_w,), jnp.int32),
            pltpu.VMEM((idx_w, _COL_DIM), jnp.float32),
            pltpu.VMEM((row_w, _COL_DIM), jnp.float32),
            pltpu.SemaphoreType.DMA,
        ],
    )
    def colmean(ids_hbm, table_hbm, out_hbm, idx_v, rows_v, acc_v, sem):
        wid = lax.axis_index("s") * nc + lax.axis_index("c")
        pltpu.sync_copy(ids_hbm.at[pl.ds(wid * idx_w, idx_w)], idx_v)
        pltpu.async_copy(table_hbm.at[idx_v], rows_v, sem).wait()

        def body(b, carry):
            acc = rows_v[b * _C, :]
            for c in range(1, _C):
                acc = acc + rows_v[b * _C + c, :]
            acc_v[b, :] = acc * (1.0 / _C)
            return carry

        lax.fori_loop(0, row_w, body, 0)
        pltpu.sync_copy(acc_v, out_hbm.at[pl.ds(wid * row_w, row_w)])

    return colmean


def _dense_body(opid_ref, stats_ref, pred_ref, cstat_ref, cemb_ref,
                optab_ref, w1t_ref, b1_ref, w2t_ref, b2_ref,
                wct_ref, bc_ref, wot_ref, bo_ref, out_ref):
    f32 = jnp.float32
    opid = opid_ref[...]
    iota = lax.broadcasted_iota(jnp.int32, (_BLK, _OP_VOCAB), 1)
    onehot = (iota == opid).astype(f32)
    op_vec = jnp.dot(onehot, optab_ref[...], preferred_element_type=f32)

    h = jnp.dot(stats_ref[...], w1t_ref[...], preferred_element_type=f32)
    h = jnp.maximum(h + b1_ref[...], 0.0)
    h = jnp.dot(h, w2t_ref[...], preferred_element_type=f32) + b2_ref[...]

    wc_big = jnp.concatenate([wct_ref[...]] * _C, axis=0) * (1.0 / _C)
    cs = jnp.dot(cstat_ref[...], wc_big, preferred_element_type=f32) + bc_ref[...]

    z = jnp.concatenate([op_vec, h, pred_ref[...], cemb_ref[...], cs], axis=-1)
    out_ref[...] = jnp.dot(z, wot_ref[...], preferred_element_type=f32) + bo_ref[...]


def _dense_call(opid2, stats, pred, cstat2, cemb, optab, w1t, b1r, w2t, b2r,
                wct, bcr, wot, bor):
    def row_spec(d):
        return pl.BlockSpec((_BLK, d), lambda i: (i, 0))

    def full_spec(a):
        return pl.BlockSpec(a.shape, lambda i: (0, 0))

    return pl.pallas_call(
        _dense_body,
        grid=(_GRID,),
        in_specs=[
            row_spec(1),
            row_spec(4),
            row_spec(_PRED_DIM),
            row_spec(_C * 4),
            row_spec(_COL_DIM),
            full_spec(optab),
            full_spec(w1t), full_spec(b1r),
            full_spec(w2t), full_spec(b2r),
            full_spec(wct), full_spec(bcr),
            full_spec(wot), full_spec(bor),
        ],
        out_specs=row_spec(_OUT_DIM),
        out_shape=jax.ShapeDtypeStruct((_B, _OUT_DIM), jnp.float32),
    )(opid2, stats, pred, cstat2, cemb, optab, w1t, b1r, w2t, b2r,
      wct, bcr, wot, bor)


def kernel(op_idx, stats, pred_flags, col_ids, col_stats,
           op_table, col_table, W1, b1, W2, b2, Wc, bc, Wo, bo):
    col_emb = _build_colmean()(col_ids.reshape(-1), col_table)
    return _dense_call(
        op_idx.reshape(_B, 1), stats, pred_flags,
        col_stats.reshape(_B, _C * 4), col_emb,
        op_table, W1.T, b1.reshape(1, -1), W2.T, b2.reshape(1, -1),
        Wc.T, bc.reshape(1, -1), Wo.T, bo.reshape(1, -1))
